# baseline (speedup 1.0000x reference)
import functools

import numpy as np
import jax
import jax.numpy as jnp
from jax import lax
from jax.experimental import pallas as pl
from jax.experimental.pallas import tpu as pltpu
from jax.experimental.pallas import tpu_sc as plsc

D_MODEL = 64
SEQ_LEN = 200
NUM_CORES = 2
NUM_SUBCORES = 16
NUM_WORKERS = NUM_CORES * NUM_SUBCORES
LANES = 16
BL = 128
SR = 8
SG = SEQ_LEN // SR
NBUF = 4
LA = 2


def _positional_encoding(length, dim):
    half = dim // 2
    posn = np.arange(length).reshape(length, 1).astype(np.float32)
    dims = np.arange(half).reshape(1, half).astype(np.float32) / half
    enc = posn / (10000.0 ** dims)
    enc = np.concatenate([np.sin(enc), np.cos(enc)], axis=-1)
    return jnp.asarray(enc, dtype=jnp.float32)


_PE = _positional_encoding(SEQ_LEN, D_MODEL)


@functools.lru_cache(maxsize=None)
def _build(batch):
    n_blocks = batch // BL
    assert n_blocks == NUM_WORKERS
    mesh = plsc.VectorSubcoreMesh(core_axis_name="c", subcore_axis_name="s")

    @functools.partial(
        pl.kernel,
        out_type=jax.ShapeDtypeStruct(
            (SEQ_LEN, D_MODEL // 8, n_blocks, 8, BL), jnp.float32
        ),
        mesh=mesh,
        scratch_types=[
            pltpu.VMEM((SG, SR, BL), jnp.int32),
            pltpu.VMEM((SEQ_LEN, D_MODEL), jnp.float32),
        ]
        + [pltpu.VMEM((BL, D_MODEL), jnp.float32) for _ in range(NBUF)]
        + [pltpu.VMEM((D_MODEL // 8, 8, BL), jnp.float32) for _ in range(NBUF)]
        + [pltpu.SemaphoreType.DMA for _ in range(2 * NBUF)],
        compiler_params=pltpu.CompilerParams(
            use_tc_tiling_on_sc=False, needs_layout_passes=False
        ),
    )
    def body(ids_hbm, table_hbm, pe_hbm, out_hbm, idx_v, pe_v, *bufs_and_sems):
        rows = list(bufs_and_sems[:NBUF])
        tiles = list(bufs_and_sems[NBUF : 2 * NBUF])
        sem_g = list(bufs_and_sems[2 * NBUF : 3 * NBUF])
        sem_w = list(bufs_and_sems[3 * NBUF : 4 * NBUF])

        wid = lax.axis_index("s") * NUM_CORES + lax.axis_index("c")
        pltpu.sync_copy(pe_hbm, pe_v)
        pltpu.sync_copy(ids_hbm.at[:, wid], idx_v)

        ii = lax.iota(jnp.int32, 16)
        dg_idx = []
        dr_idx = []
        for dgg in range(D_MODEL // LANES):
            d16 = ii + dgg * LANES
            dg_idx.append(d16 >> 3)
            dr_idx.append(d16 & 7)

        def gather_start(sg, sr, b):
            pltpu.async_copy(table_hbm.at[idx_v.at[sg, sr]], rows[b], sem_g[b])

        def gather_wait(sg, sr, b):
            pltpu.make_async_copy(
                table_hbm.at[idx_v.at[sg, sr]], rows[b], sem_g[b]
            ).wait()

        def wb_start(step, b):
            pltpu.async_copy(tiles[b], out_hbm.at[step, :, wid], sem_w[b])

        def wb_wait(step, b):
            pltpu.make_async_copy(
                tiles[b], out_hbm.at[step, :, wid], sem_w[b]
            ).wait()

        def compute(step, b):
            pe_regs = [
                pe_v[step, pl.ds(dgg * LANES, LANES)]
                for dgg in range(D_MODEL // LANES)
            ]

            def bl_body(bl, carry):
                bl_vec = jnp.full((16,), 0, jnp.int32) + bl
                for dgg in range(D_MODEL // LANES):
                    v = rows[b][bl, pl.ds(dgg * LANES, LANES)] * 9.0 + pe_regs[dgg]
                    plsc.store_scatter(
                        tiles[b], [dg_idx[dgg], dr_idx[dgg], bl_vec], v
                    )
                return carry

            lax.fori_loop(0, BL, bl_body, 0, unroll=8)

        def do_step(sg, sr, first_sg=False, last_sg=False):
            step = sg * SR + sr
            b = sr % NBUF
            b2 = (sr + LA) % NBUF
            sr2 = (sr + LA) % SR
            gather_wait(sg, sr, b)
            compute(step, b)
            wb_start(step, b)
            if not (last_sg and sr + LA >= SR):
                if not (first_sg and sr < LA):
                    wb_wait(step - LA, b2)
                sg2 = sg + (1 if sr + LA >= SR else 0)
                gather_start(sg2, sr2, b2)

        for step in range(LA):
            gather_start(0, step, step % NBUF)

        for sr in range(SR):
            do_step(0, sr, first_sg=True)

        def sg_body(sg, carry):
            for sr in range(SR):
                do_step(sg, sr)
            return carry

        lax.fori_loop(1, SG - 1, sg_body, 0, unroll=False)

        for sr in range(SR):
            do_step(SG - 1, sr, last_sg=True)
        for sr in range(SR - LA, SR):
            step = (SG - 1) * SR + sr
            wb_wait(step, sr % NBUF)

    return body


@jax.jit
def kernel(input_ids, table):
    batch, seq = input_ids.shape
    ids4 = jnp.transpose(
        input_ids.reshape(batch // BL, BL, seq // SR, SR), (2, 0, 3, 1)
    )
    out5 = _build(batch)(ids4, table, _PE)
    out = jnp.transpose(out5, (2, 4, 0, 1, 3)).reshape(batch, seq, D_MODEL)
    return out


# speedup vs baseline: 2.2659x; 2.2659x over previous
import functools

import numpy as np
import jax
import jax.numpy as jnp
from jax import lax
from jax.experimental import pallas as pl
from jax.experimental.pallas import tpu as pltpu
from jax.experimental.pallas import tpu_sc as plsc

D_MODEL = 64
SEQ_LEN = 200
NUM_CORES = 2
NUM_SUBCORES = 16
NUM_WORKERS = NUM_CORES * NUM_SUBCORES
LANES = 16
BL = 128
SR = 8
SG = SEQ_LEN // SR
NBUF = 4
LA = 2


def _positional_encoding(length, dim):
    half = dim // 2
    posn = np.arange(length).reshape(length, 1).astype(np.float32)
    dims = np.arange(half).reshape(1, half).astype(np.float32) / half
    enc = posn / (10000.0 ** dims)
    enc = np.concatenate([np.sin(enc), np.cos(enc)], axis=-1)
    return jnp.asarray(enc, dtype=jnp.float32)


_PE = _positional_encoding(SEQ_LEN, D_MODEL)


@functools.lru_cache(maxsize=None)
def _build(batch):
    n_blocks = batch // BL
    assert n_blocks == NUM_WORKERS
    mesh = plsc.VectorSubcoreMesh(core_axis_name="c", subcore_axis_name="s")

    @functools.partial(
        pl.kernel,
        out_type=jax.ShapeDtypeStruct(
            (SEQ_LEN, D_MODEL // 8, n_blocks, 8, BL), jnp.float32
        ),
        mesh=mesh,
        scratch_types=[
            pltpu.VMEM((SG, SR, BL), jnp.int32),
            pltpu.VMEM((SEQ_LEN, D_MODEL), jnp.float32),
        ]
        + [pltpu.VMEM((BL, D_MODEL), jnp.float32) for _ in range(NBUF)]
        + [pltpu.VMEM((D_MODEL // 8, 8, BL + 1), jnp.float32) for _ in range(NBUF)]
        + [pltpu.SemaphoreType.DMA for _ in range(2 * NBUF)],
        compiler_params=pltpu.CompilerParams(
            use_tc_tiling_on_sc=False, needs_layout_passes=False
        ),
    )
    def body(ids_hbm, table_hbm, pe_hbm, out_hbm, idx_v, pe_v, *bufs_and_sems):
        rows = list(bufs_and_sems[:NBUF])
        tiles = list(bufs_and_sems[NBUF : 2 * NBUF])
        sem_g = list(bufs_and_sems[2 * NBUF : 3 * NBUF])
        sem_w = list(bufs_and_sems[3 * NBUF : 4 * NBUF])

        wid = lax.axis_index("s") * NUM_CORES + lax.axis_index("c")
        pltpu.sync_copy(pe_hbm, pe_v)
        pltpu.sync_copy(ids_hbm.at[:, wid], idx_v)

        ii = lax.iota(jnp.int32, 16)
        dg_idx = []
        dr_idx = []
        for dgg in range(D_MODEL // LANES):
            d16 = ii + dgg * LANES
            dg_idx.append(d16 >> 3)
            dr_idx.append(d16 & 7)

        def gather_start(sg, sr, b):
            pltpu.async_copy(table_hbm.at[idx_v.at[sg, sr]], rows[b], sem_g[b])

        def gather_wait(sg, sr, b):
            pltpu.make_async_copy(
                table_hbm.at[idx_v.at[sg, sr]], rows[b], sem_g[b]
            ).wait()

        def wb_start(step, b):
            pltpu.async_copy(
                tiles[b].at[:, :, pl.ds(0, BL)], out_hbm.at[step, :, wid], sem_w[b]
            )

        def wb_wait(step, b):
            pltpu.make_async_copy(
                tiles[b].at[:, :, pl.ds(0, BL)], out_hbm.at[step, :, wid], sem_w[b]
            ).wait()

        def compute(step, b):
            pe_regs = [
                pe_v[step, pl.ds(dgg * LANES, LANES)]
                for dgg in range(D_MODEL // LANES)
            ]

            @plsc.parallel_loop(0, BL, step=1, unroll=8)
            def bl_body(bl):
                bl_vec = jnp.full((16,), 0, jnp.int32) + bl
                for dgg in range(D_MODEL // LANES):
                    v = rows[b][bl, pl.ds(dgg * LANES, LANES)] * 9.0 + pe_regs[dgg]
                    plsc.store_scatter(
                        tiles[b], [dg_idx[dgg], dr_idx[dgg], bl_vec], v
                    )

        def do_step(sg, sr, first_sg=False, last_sg=False):
            step = sg * SR + sr
            b = sr % NBUF
            b2 = (sr + LA) % NBUF
            sr2 = (sr + LA) % SR
            gather_wait(sg, sr, b)
            compute(step, b)
            wb_start(step, b)
            if not (last_sg and sr + LA >= SR):
                if not (first_sg and sr < LA):
                    wb_wait(step - LA, b2)
                sg2 = sg + (1 if sr + LA >= SR else 0)
                gather_start(sg2, sr2, b2)

        for step in range(LA):
            gather_start(0, step, step % NBUF)

        for sr in range(SR):
            do_step(0, sr, first_sg=True)

        def sg_body(sg, carry):
            for sr in range(SR):
                do_step(sg, sr)
            return carry

        lax.fori_loop(1, SG - 1, sg_body, 0, unroll=False)

        for sr in range(SR):
            do_step(SG - 1, sr, last_sg=True)
        for sr in range(SR - LA, SR):
            step = (SG - 1) * SR + sr
            wb_wait(step, sr % NBUF)

    return body


@jax.jit
def kernel(input_ids, table):
    batch, seq = input_ids.shape
    ids4 = jnp.transpose(
        input_ids.reshape(batch // BL, BL, seq // SR, SR), (2, 0, 3, 1)
    )
    out5 = _build(batch)(ids4, table, _PE)
    out = jnp.transpose(out5, (2, 4, 0, 1, 3)).reshape(batch, seq, D_MODEL)
    return out
